# Initial kernel scaffold; baseline (speedup 1.0000x reference)
import functools
import jax
import jax.numpy as jnp
from jax import lax
from jax.experimental import pallas as pl
from jax.experimental.pallas import tpu as pltpu
from jax.experimental.pallas import tpu_sc as plsc

N_NODES_K = 10000
N_EDGES_K = 160000
D_K = 256
H_K = D_K // 2
B_K = 128
NBLK = N_EDGES_K // B_K
NSUB = 16
L = 16
ITERS_PER_SUB = (NBLK + NSUB - 1) // NSUB
ROWS_PER_SUB = N_NODES_K // NSUB

_mesh = plsc.VectorSubcoreMesh(core_axis_name="c", subcore_axis_name="s")


@functools.partial(
    pl.kernel,
    out_type=jax.ShapeDtypeStruct((2, N_NODES_K, H_K), jnp.float32),
    mesh=_mesh,
    scratch_types=[
        pltpu.VMEM((B_K,), jnp.int32),
        pltpu.VMEM((B_K,), jnp.int32),
        pltpu.VMEM((B_K,), jnp.float32),
        pltpu.VMEM((B_K, H_K), jnp.float32),
        pltpu.VMEM_SHARED((N_NODES_K, H_K), jnp.float32),
        pltpu.SemaphoreType.DMA,
        pltpu.SemaphoreType.DMA,
    ],
)
def _spmm_sc(row_h, col_h, val_h, oa_h, ob_h, out_h,
             colb, rowb, valb, msg, acc, sem_g, sem_i):
    c = lax.axis_index("c")
    s = lax.axis_index("s")
    zeros16 = jnp.zeros((L,), jnp.float32)

    @pl.loop(0, B_K)
    def _(r):
        for j in range(H_K // L):
            msg[r, pl.ds(j * L, L)] = zeros16

    r0 = s * ROWS_PER_SUB
    n_full = ROWS_PER_SUB // B_K
    rem = ROWS_PER_SUB - n_full * B_K
    for kk in range(n_full):
        pltpu.sync_copy(msg, acc.at[pl.ds(r0 + kk * B_K, B_K), :])
    pltpu.sync_copy(msg.at[pl.ds(0, rem), :],
                    acc.at[pl.ds(r0 + n_full * B_K, rem), :])
    plsc.subcore_barrier()

    @pl.loop(0, ITERS_PER_SUB)
    def _(k):
        b = s + k * NSUB

        @pl.when(b < NBLK)
        def _():
            base = b * B_K
            d1 = pltpu.async_copy(col_h.at[pl.ds(base, B_K)], colb, sem_i)
            d2 = pltpu.async_copy(row_h.at[pl.ds(base, B_K)], rowb, sem_i)
            d3 = pltpu.async_copy(val_h.at[pl.ds(base, B_K)], valb, sem_i)
            d1.wait()
            d2.wait()
            d3.wait()

            @pl.when(c == 0)
            def _():
                pltpu.async_copy(oa_h.at[colb], msg, sem_g).wait()

            @pl.when(c == 1)
            def _():
                pltpu.async_copy(ob_h.at[colb], msg, sem_g).wait()

            @pl.loop(0, B_K // L)
            def _(g):
                vals16 = valb[pl.ds(g * L, L)]
                for i in range(L):
                    vspl = jnp.take_along_axis(
                        vals16, jnp.full((L,), i, jnp.int32), axis=0)
                    e = g * L + i
                    for j in range(H_K // L):
                        sl = pl.ds(j * L, L)
                        msg[e, sl] = msg[e, sl] * vspl

            pltpu.sync_copy(msg, acc.at[rowb], add=True)

    plsc.subcore_barrier()

    for kk in range(n_full):
        pltpu.sync_copy(acc.at[pl.ds(r0 + kk * B_K, B_K), :],
                        out_h.at[c, pl.ds(r0 + kk * B_K, B_K), :])
    pltpu.sync_copy(acc.at[pl.ds(r0 + n_full * B_K, rem), :],
                    out_h.at[c, pl.ds(r0 + n_full * B_K, rem), :])


def kernel(row, col, value, other):
    oa = other[:, :H_K]
    ob = other[:, H_K:]
    out2 = _spmm_sc(row, col, value, oa, ob)
    return out2.transpose(1, 0, 2).reshape(N_NODES_K, D_K)


# speedup vs baseline: 4.2073x; 4.2073x over previous
import functools
import jax
import jax.numpy as jnp
from jax import lax
from jax.experimental import pallas as pl
from jax.experimental.pallas import tpu as pltpu
from jax.experimental.pallas import tpu_sc as plsc

N_NODES_K = 10000
N_EDGES_K = 160000
D_K = 256
H_K = D_K // 2
B_K = 128
NBLK = N_EDGES_K // B_K
NSUB = 16
L = 16
ITERS_PER_SUB = (NBLK + NSUB - 1) // NSUB
ROWS_PER_SUB = 624

_mesh = plsc.VectorSubcoreMesh(core_axis_name="c", subcore_axis_name="s")


@functools.partial(
    pl.kernel,
    out_type=jax.ShapeDtypeStruct((2, N_NODES_K, H_K), jnp.float32),
    mesh=_mesh,
    scratch_types=[
        pltpu.VMEM((B_K,), jnp.int32),
        pltpu.VMEM((B_K,), jnp.int32),
        pltpu.VMEM((B_K,), jnp.float32),
        pltpu.VMEM((B_K, H_K), jnp.float32),
        pltpu.VMEM_SHARED((N_NODES_K, H_K), jnp.float32),
        pltpu.SemaphoreType.DMA,
        pltpu.SemaphoreType.DMA,
    ],
)
def _spmm_sc(row_h, col_h, val_h, oa_h, ob_h, out_h,
             colb, rowb, valb, msg, acc, sem_g, sem_i):
    c = lax.axis_index("c")
    s = lax.axis_index("s")
    zeros16 = jnp.zeros((L,), jnp.float32)

    @pl.loop(0, B_K)
    def _(r):
        for j in range(H_K // L):
            msg[r, pl.ds(j * L, L)] = zeros16

    r0 = s * ROWS_PER_SUB
    for kk in range(4):
        pltpu.sync_copy(msg, acc.at[pl.ds(r0 + kk * B_K, B_K), :])

    @pl.when(s < NSUB - 1)
    def _():
        pltpu.sync_copy(msg.at[pl.ds(0, 112), :],
                        acc.at[pl.ds(r0 + 4 * B_K, 112), :])

    @pl.when(s == NSUB - 1)
    def _():
        pltpu.sync_copy(msg, acc.at[pl.ds(r0 + 4 * B_K, B_K), :])

    plsc.subcore_barrier()

    @pl.loop(0, ITERS_PER_SUB)
    def _(k):
        b = s + k * NSUB

        @pl.when(b < NBLK)
        def _():
            base = b * B_K
            d1 = pltpu.async_copy(col_h.at[pl.ds(base, B_K)], colb, sem_i)
            d2 = pltpu.async_copy(row_h.at[pl.ds(base, B_K)], rowb, sem_i)
            d3 = pltpu.async_copy(val_h.at[pl.ds(base, B_K)], valb, sem_i)
            d1.wait()
            d2.wait()
            d3.wait()

            @pl.when(c == 0)
            def _():
                pltpu.async_copy(oa_h.at[colb], msg, sem_g).wait()

            @pl.when(c == 1)
            def _():
                pltpu.async_copy(ob_h.at[colb], msg, sem_g).wait()

            @pl.loop(0, B_K // L)
            def _(g):
                vals16 = valb[pl.ds(g * L, L)]
                for i in range(L):
                    vspl = jnp.take_along_axis(
                        vals16, jnp.full((L,), i, jnp.int32), axis=0)
                    e = g * L + i
                    for j in range(H_K // L):
                        sl = pl.ds(j * L, L)
                        msg[e, sl] = msg[e, sl] * vspl

            pltpu.sync_copy(msg, acc.at[rowb], add=True)

    plsc.subcore_barrier()

    for kk in range(4):
        pltpu.sync_copy(acc.at[pl.ds(r0 + kk * B_K, B_K), :],
                        out_h.at[c, pl.ds(r0 + kk * B_K, B_K), :])

    @pl.when(s < NSUB - 1)
    def _():
        pltpu.sync_copy(acc.at[pl.ds(r0 + 4 * B_K, 112), :],
                        out_h.at[c, pl.ds(r0 + 4 * B_K, 112), :])

    @pl.when(s == NSUB - 1)
    def _():
        pltpu.sync_copy(acc.at[pl.ds(r0 + 4 * B_K, B_K), :],
                        out_h.at[c, pl.ds(r0 + 4 * B_K, B_K), :])


def kernel(row, col, value, other):
    oa = other[:, :H_K]
    ob = other[:, H_K:]
    out2 = _spmm_sc(row, col, value, oa, ob)
    return out2.transpose(1, 0, 2).reshape(N_NODES_K, D_K)
